# baseline (speedup 1.0000x reference)
import functools

import jax
import jax.numpy as jnp
from jax import lax
from jax.experimental import pallas as pl
from jax.experimental.pallas import tpu as pltpu
from jax.experimental.pallas import tpu_sc as plsc

N = 10000
D = 128
E = 320000
NC = 2
NS = 16
NW = NC * NS
C = 128
NCHUNK = 80
HCHUNK = NCHUNK // 2
EPP = NCHUNK * C
EPAD = NW * EPP
NPAD = 10240
RPS = NPAD // NS
PAD_SRC = NPAD - 1
PAD_DST = NPAD - 2

_mesh = plsc.VectorSubcoreMesh(
    core_axis_name="c", subcore_axis_name="s", num_cores=NC, num_subcores=NS)
_params = pltpu.CompilerParams(use_tc_tiling_on_sc=True)


@functools.partial(
    pl.kernel,
    out_type=jax.ShapeDtypeStruct((NC * 2 * NPAD,), jnp.float32),
    mesh=_mesh,
    compiler_params=_params,
    scratch_types=[
        pltpu.VMEM((NCHUNK, C), jnp.int32),
        pltpu.VMEM((NCHUNK, C), jnp.int32),
        pltpu.VMEM((C,), jnp.float32),
        pltpu.VMEM((RPS,), jnp.float32),
        pltpu.VMEM_SHARED((NPAD,), jnp.float32),
        pltpu.VMEM_SHARED((NPAD,), jnp.float32),
    ],
)
def _degrees(src_hbm, dst_hbm, out_hbm, src_v, dst_v, ones_v, zz_v,
             dsrc_sh, ddst_sh):
    cid = lax.axis_index("c")
    sid = lax.axis_index("s")
    wid = cid * NS + sid
    for i in range(C // 16):
        ones_v[pl.ds(i * 16, 16)] = jnp.ones((16,), jnp.float32)

    def zfill(i, carry):
        zz_v[pl.ds(i * 16, 16)] = jnp.zeros((16,), jnp.float32)
        return carry

    lax.fori_loop(0, RPS // 16, zfill, 0)
    pltpu.sync_copy(zz_v, dsrc_sh.at[pl.ds(sid * RPS, RPS)])
    pltpu.sync_copy(zz_v, ddst_sh.at[pl.ds(sid * RPS, RPS)])
    pltpu.sync_copy(src_hbm.at[wid], src_v)
    pltpu.sync_copy(dst_hbm.at[wid], dst_v)
    plsc.subcore_barrier()

    def body(j, carry):
        pltpu.sync_copy(ones_v, dsrc_sh.at[src_v.at[j]], add=True)
        pltpu.sync_copy(ones_v, ddst_sh.at[dst_v.at[j]], add=True)
        return carry

    lax.fori_loop(0, NCHUNK, body, 0)
    plsc.subcore_barrier()
    sl = pl.ds(sid * RPS, RPS)
    pltpu.sync_copy(dsrc_sh.at[sl],
                    out_hbm.at[pl.ds(cid * 2 * NPAD + sid * RPS, RPS)])
    pltpu.sync_copy(ddst_sh.at[sl],
                    out_hbm.at[pl.ds(cid * 2 * NPAD + NPAD + sid * RPS, RPS)])


@functools.partial(
    pl.kernel,
    out_type=jax.ShapeDtypeStruct((NC * NPAD, D), jnp.float32),
    mesh=_mesh,
    compiler_params=_params,
    scratch_types=[
        pltpu.VMEM((HCHUNK, C), jnp.int32),
        pltpu.VMEM((HCHUNK, C), jnp.int32),
        pltpu.VMEM((C, D), jnp.float32),
        pltpu.VMEM((C, D), jnp.float32),
        pltpu.VMEM_SHARED((NPAD, D), jnp.float32),
        pltpu.SemaphoreType.DMA,
        pltpu.SemaphoreType.DMA,
    ],
)
def _propagate(g_hbm, src_hbm, dst_hbm, out_hbm, src_v, dst_v, rows0, rows1,
               acc_sh, sem0, sem1):
    cid = lax.axis_index("c")
    sid = lax.axis_index("s")
    wid = cid * NS + sid
    sl = pl.ds(sid * RPS, RPS)
    for i in range(C // 16):
        rows0[pl.ds(i * 16, 16), :] = jnp.zeros((16, D), jnp.float32)
    for r in range(RPS // C):
        pltpu.sync_copy(rows0, acc_sh.at[pl.ds(sid * RPS + r * C, C)])
    plsc.subcore_barrier()

    def fire(j, rows, sem):
        pltpu.make_async_copy(g_hbm.at[src_v.at[j]], rows, sem).start()

    def drain(j, rows, sem):
        pltpu.make_async_copy(g_hbm.at[src_v.at[j]], rows, sem).wait()
        pltpu.sync_copy(rows, acc_sh.at[dst_v.at[j]], add=True)

    for h in range(2):
        hs = pl.ds(h * HCHUNK, HCHUNK)
        pltpu.sync_copy(src_hbm.at[wid, hs], src_v)
        pltpu.sync_copy(dst_hbm.at[wid, hs], dst_v)
        fire(0, rows0, sem0)
        fire(1, rows1, sem1)

        def body(i, carry):
            j = 2 * i
            drain(j, rows0, sem0)
            fire(j + 2, rows0, sem0)
            drain(j + 1, rows1, sem1)
            fire(j + 3, rows1, sem1)
            return carry

        lax.fori_loop(0, HCHUNK // 2 - 1, body, 0)
        drain(HCHUNK - 2, rows0, sem0)
        drain(HCHUNK - 1, rows1, sem1)
    plsc.subcore_barrier()
    pltpu.sync_copy(acc_sh.at[sl], out_hbm.at[pl.ds(cid * NPAD + sid * RPS, RPS)])


def _norms_prescale(degs, x):
    def body(deg_ref, x_ref, nout_ref, nin_ref, g0_ref):
        d_out = deg_ref[0] + deg_ref[2]
        d_in = deg_ref[1] + deg_ref[3]
        n_out = lax.rsqrt(jnp.maximum(d_out, 1.0))
        n_in = lax.rsqrt(jnp.maximum(d_in, 1.0))
        nout_ref[...] = n_out
        nin_ref[...] = n_in
        g0_ref[pl.ds(0, N), :] = x_ref[...] * n_out[:N]
        g0_ref[pl.ds(N, NPAD - N), :] = jnp.zeros((NPAD - N, D), jnp.float32)

    return pl.pallas_call(
        body,
        out_shape=[
            jax.ShapeDtypeStruct((NPAD, 1), jnp.float32),
            jax.ShapeDtypeStruct((NPAD, 1), jnp.float32),
            jax.ShapeDtypeStruct((NPAD, D), jnp.float32),
        ],
    )(degs, x)


def _layer_mid(p, n_in, n_out, w, b):
    def body(p_ref, nin_ref, nout_ref, w_ref, b_ref, h_ref, g_ref):
        agg = (p_ref[pl.ds(0, NPAD), :] + p_ref[pl.ds(NPAD, NPAD), :]) * nin_ref[...]
        t = jnp.dot(agg, w_ref[...], preferred_element_type=jnp.float32)
        h = jnp.maximum(t + b_ref[...], 0.0)
        h_ref[...] = h[:N]
        g_ref[...] = h * nout_ref[...]

    return pl.pallas_call(
        body,
        out_shape=[
            jax.ShapeDtypeStruct((N, D), jnp.float32),
            jax.ShapeDtypeStruct((NPAD, D), jnp.float32),
        ],
    )(p, n_in, n_out, w, b)


def _layer_out(p, n_in, w, b):
    def body(p_ref, nin_ref, w_ref, b_ref, h_ref):
        agg = (p_ref[pl.ds(0, NPAD), :] + p_ref[pl.ds(NPAD, NPAD), :]) * nin_ref[...]
        t = jnp.dot(agg, w_ref[...], preferred_element_type=jnp.float32)
        h_ref[...] = t[:N] + b_ref[...]

    return pl.pallas_call(
        body,
        out_shape=jax.ShapeDtypeStruct((N, D), jnp.float32),
    )(p, n_in, w, b)


def kernel(x, edge_index, W1, b1, W2, b2):
    pad_s = jnp.full((EPAD - E,), PAD_SRC, jnp.int32)
    pad_d = jnp.full((EPAD - E,), PAD_DST, jnp.int32)
    src3 = jnp.concatenate([edge_index[0], pad_s]).reshape(NW, NCHUNK, C)
    dst3 = jnp.concatenate([edge_index[1], pad_d]).reshape(NW, NCHUNK, C)
    degs = _degrees(src3, dst3).reshape(NC * 2, NPAD, 1)
    n_out, n_in, g0 = _norms_prescale(degs, x)
    b1r = b1.reshape(1, D)
    b2r = b2.reshape(1, D)
    p = _propagate(g0, src3, dst3)
    h1, g1 = _layer_mid(p, n_in, n_out, W1, b1r)
    q = _propagate(g1, src3, dst3)
    h2 = _layer_out(q, n_in, W2, b2r)
    return (h1, h2)


# speedup vs baseline: 3.0079x; 3.0079x over previous
import functools

import jax
import jax.numpy as jnp
from jax import lax
from jax.experimental import pallas as pl
from jax.experimental.pallas import tpu as pltpu
from jax.experimental.pallas import tpu_sc as plsc

N = 10000
D = 128
E = 320000
NC = 2
NS = 16
NW = NC * NS
EP = E // NW
C = 100
NCHUNK = EP // C
CD = 80
NCHUNKD = EP // CD
NPAD = 10240
RPS = NPAD // NS
assert E == NW * NCHUNK * C and E == NW * NCHUNKD * CD and NCHUNK % 2 == 0

_mesh = plsc.VectorSubcoreMesh(
    core_axis_name="c", subcore_axis_name="s", num_cores=NC, num_subcores=NS)


@functools.partial(
    pl.kernel,
    out_type=jax.ShapeDtypeStruct((NC * 2 * NPAD,), jnp.float32),
    mesh=_mesh,
    compiler_params=pltpu.CompilerParams(use_tc_tiling_on_sc=False),
    scratch_types=[
        pltpu.VMEM((NCHUNKD, CD), jnp.int32),
        pltpu.VMEM((NCHUNKD, CD), jnp.int32),
        pltpu.VMEM((CD,), jnp.float32),
        pltpu.VMEM((RPS,), jnp.float32),
        pltpu.VMEM_SHARED((NPAD,), jnp.float32),
        pltpu.VMEM_SHARED((NPAD,), jnp.float32),
    ],
)
def _degrees(src_hbm, dst_hbm, out_hbm, src_v, dst_v, ones_v, zz_v,
             dsrc_sh, ddst_sh):
    cid = lax.axis_index("c")
    sid = lax.axis_index("s")
    wid = cid * NS + sid
    for i in range(CD // 16):
        ones_v[pl.ds(i * 16, 16)] = jnp.ones((16,), jnp.float32)

    def zfill(i, carry):
        zz_v[pl.ds(i * 16, 16)] = jnp.zeros((16,), jnp.float32)
        return carry

    lax.fori_loop(0, RPS // 16, zfill, 0)
    pltpu.sync_copy(zz_v, dsrc_sh.at[pl.ds(sid * RPS, RPS)])
    pltpu.sync_copy(zz_v, ddst_sh.at[pl.ds(sid * RPS, RPS)])
    pltpu.sync_copy(src_hbm.at[wid], src_v)
    pltpu.sync_copy(dst_hbm.at[wid], dst_v)
    plsc.subcore_barrier()

    def body(j, carry):
        pltpu.sync_copy(ones_v, dsrc_sh.at[src_v.at[j]], add=True)
        pltpu.sync_copy(ones_v, ddst_sh.at[dst_v.at[j]], add=True)
        return carry

    lax.fori_loop(0, NCHUNKD, body, 0)
    plsc.subcore_barrier()
    sl = pl.ds(sid * RPS, RPS)
    pltpu.sync_copy(dsrc_sh.at[sl],
                    out_hbm.at[pl.ds(cid * 2 * NPAD + sid * RPS, RPS)])
    pltpu.sync_copy(ddst_sh.at[sl],
                    out_hbm.at[pl.ds(cid * 2 * NPAD + NPAD + sid * RPS, RPS)])


@functools.partial(
    pl.kernel,
    out_type=jax.ShapeDtypeStruct((NC * NPAD, D), jnp.float32),
    mesh=_mesh,
    compiler_params=pltpu.CompilerParams(use_tc_tiling_on_sc=False),
    scratch_types=[
        pltpu.VMEM((NCHUNK, C), jnp.int32),
        pltpu.VMEM((NCHUNK, C), jnp.int32),
        pltpu.VMEM((C, D), jnp.float32),
        pltpu.VMEM((C, D), jnp.float32),
        pltpu.VMEM((16, D), jnp.float32),
        pltpu.VMEM_SHARED((NPAD, D), jnp.float32),
        pltpu.SemaphoreType.DMA,
        pltpu.SemaphoreType.DMA,
    ],
)
def _propagate(g_hbm, src_hbm, dst_hbm, out_hbm, src_v, dst_v, rows0, rows1,
               zz_v, acc_sh, sem0, sem1):
    cid = lax.axis_index("c")
    sid = lax.axis_index("s")
    wid = cid * NS + sid
    sl = pl.ds(sid * RPS, RPS)
    zz_v[...] = jnp.zeros((16, D), jnp.float32)

    def zbody(r, carry):
        pltpu.sync_copy(zz_v, acc_sh.at[pl.ds(sid * RPS + r * 16, 16)])
        return carry

    lax.fori_loop(0, RPS // 16, zbody, 0)
    pltpu.sync_copy(src_hbm.at[wid], src_v)
    pltpu.sync_copy(dst_hbm.at[wid], dst_v)
    plsc.subcore_barrier()

    def fire(j, rows, sem):
        pltpu.make_async_copy(g_hbm.at[src_v.at[j]], rows, sem).start()

    def drain(j, rows, sem):
        pltpu.make_async_copy(g_hbm.at[src_v.at[j]], rows, sem).wait()
        pltpu.sync_copy(rows, acc_sh.at[dst_v.at[j]], add=True)

    fire(0, rows0, sem0)
    fire(1, rows1, sem1)

    def body(i, carry):
        j = 2 * i
        drain(j, rows0, sem0)
        fire(j + 2, rows0, sem0)
        drain(j + 1, rows1, sem1)
        fire(j + 3, rows1, sem1)
        return carry

    lax.fori_loop(0, NCHUNK // 2 - 1, body, 0)
    drain(NCHUNK - 2, rows0, sem0)
    drain(NCHUNK - 1, rows1, sem1)
    plsc.subcore_barrier()
    pltpu.sync_copy(acc_sh.at[sl], out_hbm.at[pl.ds(cid * NPAD + sid * RPS, RPS)])


BR = 1024
NBLK = -(-N // BR)


def _norms_prescale(degs, x):
    def body(deg_ref, x_ref, nout_ref, nin_ref, g0_ref):
        d_out = deg_ref[0] + deg_ref[2]
        d_in = deg_ref[1] + deg_ref[3]
        n_out = lax.rsqrt(jnp.maximum(d_out, 1.0))
        n_in = lax.rsqrt(jnp.maximum(d_in, 1.0))
        nout_ref[...] = n_out
        nin_ref[...] = n_in
        g0_ref[...] = x_ref[...] * n_out

    return pl.pallas_call(
        body,
        grid=(NBLK,),
        in_specs=[
            pl.BlockSpec((4, BR, 1), lambda i: (0, i, 0)),
            pl.BlockSpec((BR, D), lambda i: (i, 0)),
        ],
        out_specs=[
            pl.BlockSpec((BR, 1), lambda i: (i, 0)),
            pl.BlockSpec((BR, 1), lambda i: (i, 0)),
            pl.BlockSpec((BR, D), lambda i: (i, 0)),
        ],
        out_shape=[
            jax.ShapeDtypeStruct((N, 1), jnp.float32),
            jax.ShapeDtypeStruct((N, 1), jnp.float32),
            jax.ShapeDtypeStruct((N, D), jnp.float32),
        ],
    )(degs, x)


def _layer_mid(p, n_in, n_out, w, b):
    def body(p0_ref, p1_ref, nin_ref, nout_ref, w_ref, b_ref, h_ref, g_ref):
        agg = (p0_ref[...] + p1_ref[...]) * nin_ref[...]
        t = jnp.dot(agg, w_ref[...], preferred_element_type=jnp.float32)
        h = jnp.maximum(t + b_ref[...], 0.0)
        h_ref[...] = h
        g_ref[...] = h * nout_ref[...]

    return pl.pallas_call(
        body,
        grid=(NBLK,),
        in_specs=[
            pl.BlockSpec((BR, D), lambda i: (i, 0)),
            pl.BlockSpec((BR, D), lambda i: (i + NPAD // BR, 0)),
            pl.BlockSpec((BR, 1), lambda i: (i, 0)),
            pl.BlockSpec((BR, 1), lambda i: (i, 0)),
            pl.BlockSpec((D, D), lambda i: (0, 0)),
            pl.BlockSpec((1, D), lambda i: (0, 0)),
        ],
        out_specs=[
            pl.BlockSpec((BR, D), lambda i: (i, 0)),
            pl.BlockSpec((BR, D), lambda i: (i, 0)),
        ],
        out_shape=[
            jax.ShapeDtypeStruct((N, D), jnp.float32),
            jax.ShapeDtypeStruct((N, D), jnp.float32),
        ],
    )(p, p, n_in, n_out, w, b)


def _layer_out(p, n_in, w, b):
    def body(p0_ref, p1_ref, nin_ref, w_ref, b_ref, h_ref):
        agg = (p0_ref[...] + p1_ref[...]) * nin_ref[...]
        t = jnp.dot(agg, w_ref[...], preferred_element_type=jnp.float32)
        h_ref[...] = t + b_ref[...]

    return pl.pallas_call(
        body,
        grid=(NBLK,),
        in_specs=[
            pl.BlockSpec((BR, D), lambda i: (i, 0)),
            pl.BlockSpec((BR, D), lambda i: (i + NPAD // BR, 0)),
            pl.BlockSpec((BR, 1), lambda i: (i, 0)),
            pl.BlockSpec((D, D), lambda i: (0, 0)),
            pl.BlockSpec((1, D), lambda i: (0, 0)),
        ],
        out_specs=pl.BlockSpec((BR, D), lambda i: (i, 0)),
        out_shape=jax.ShapeDtypeStruct((N, D), jnp.float32),
    )(p, p, n_in, w, b)


def kernel(x, edge_index, W1, b1, W2, b2):
    src3 = edge_index[0].reshape(NW, NCHUNK, C)
    dst3 = edge_index[1].reshape(NW, NCHUNK, C)
    src3d = edge_index[0].reshape(NW, NCHUNKD, CD)
    dst3d = edge_index[1].reshape(NW, NCHUNKD, CD)
    degs = _degrees(src3d, dst3d).reshape(NC * 2, NPAD, 1)
    n_out, n_in, g0 = _norms_prescale(degs, x)
    b1r = b1.reshape(1, D)
    b2r = b2.reshape(1, D)
    p = _propagate(g0, src3, dst3)
    h1, g1 = _layer_mid(p, n_in, n_out, W1, b1r)
    q = _propagate(g1, src3, dst3)
    h2 = _layer_out(q, n_in, W2, b2r)
    return (h1, h2)


# speedup vs baseline: 3.0631x; 1.0184x over previous
import functools

import jax
import jax.numpy as jnp
from jax import lax
from jax.experimental import pallas as pl
from jax.experimental.pallas import tpu as pltpu
from jax.experimental.pallas import tpu_sc as plsc

N = 10000
D = 128
E = 320000
NC = 2
NS = 16
NW = NC * NS
EP = E // NW
C = 100
NCHUNK = EP // C
CDCH = 160
CDC = 128
NPAD = 10240
RPS = NPAD // NS
assert E == NW * NCHUNK * C and NCHUNK % 2 == 0
assert NW * CDCH * CDC == 2 * (E + 7680)

_mesh = plsc.VectorSubcoreMesh(
    core_axis_name="c", subcore_axis_name="s", num_cores=NC, num_subcores=NS)


@functools.partial(
    pl.kernel,
    out_type=jax.ShapeDtypeStruct((NC * 2 * NPAD,), jnp.float32),
    mesh=_mesh,
    compiler_params=pltpu.CompilerParams(use_tc_tiling_on_sc=False),
    scratch_types=[
        pltpu.VMEM((CDCH, CDC), jnp.int32),
        pltpu.VMEM((CDC,), jnp.float32),
        pltpu.VMEM((2 * RPS,), jnp.float32),
        pltpu.VMEM_SHARED((2 * NPAD,), jnp.float32),
    ],
)
def _degrees(idx_hbm, out_hbm, idx_v, ones_v, zz_v, hist_sh):
    cid = lax.axis_index("c")
    sid = lax.axis_index("s")
    wid = cid * NS + sid
    for i in range(CDC // 16):
        ones_v[pl.ds(i * 16, 16)] = jnp.ones((16,), jnp.float32)

    def zfill(i, carry):
        zz_v[pl.ds(i * 16, 16)] = jnp.zeros((16,), jnp.float32)
        return carry

    lax.fori_loop(0, 2 * RPS // 16, zfill, 0)
    pltpu.sync_copy(zz_v, hist_sh.at[pl.ds(sid * 2 * RPS, 2 * RPS)])
    pltpu.sync_copy(idx_hbm.at[wid], idx_v)
    plsc.subcore_barrier()

    def body(j, carry):
        pltpu.sync_copy(ones_v, hist_sh.at[idx_v.at[j]], add=True)
        return carry

    lax.fori_loop(0, CDCH, body, 0)
    plsc.subcore_barrier()
    sl = pl.ds(sid * 2 * RPS, 2 * RPS)
    pltpu.sync_copy(hist_sh.at[sl],
                    out_hbm.at[pl.ds(cid * 2 * NPAD + sid * 2 * RPS, 2 * RPS)])


@functools.partial(
    pl.kernel,
    out_type=jax.ShapeDtypeStruct((NC * NPAD, D), jnp.float32),
    mesh=_mesh,
    compiler_params=pltpu.CompilerParams(use_tc_tiling_on_sc=False),
    scratch_types=[
        pltpu.VMEM((NCHUNK, C), jnp.int32),
        pltpu.VMEM((NCHUNK, C), jnp.int32),
        pltpu.VMEM((C, D), jnp.float32),
        pltpu.VMEM((C, D), jnp.float32),
        pltpu.VMEM((16, D), jnp.float32),
        pltpu.VMEM_SHARED((NPAD, D), jnp.float32),
        pltpu.SemaphoreType.DMA,
        pltpu.SemaphoreType.DMA,
    ],
)
def _propagate(g_hbm, src_hbm, dst_hbm, out_hbm, src_v, dst_v, rows0, rows1,
               zz_v, acc_sh, sem0, sem1):
    cid = lax.axis_index("c")
    sid = lax.axis_index("s")
    wid = cid * NS + sid
    sl = pl.ds(sid * RPS, RPS)
    zz_v[...] = jnp.zeros((16, D), jnp.float32)

    def zbody(r, carry):
        pltpu.sync_copy(zz_v, acc_sh.at[pl.ds(sid * RPS + r * 16, 16)])
        return carry

    lax.fori_loop(0, RPS // 16, zbody, 0)
    pltpu.sync_copy(src_hbm.at[wid], src_v)
    pltpu.sync_copy(dst_hbm.at[wid], dst_v)
    plsc.subcore_barrier()

    def fire(j, rows, sem):
        pltpu.make_async_copy(g_hbm.at[src_v.at[j]], rows, sem).start()

    def drain(j, rows, sem):
        pltpu.make_async_copy(g_hbm.at[src_v.at[j]], rows, sem).wait()
        pltpu.sync_copy(rows, acc_sh.at[dst_v.at[j]], add=True)

    fire(0, rows0, sem0)
    fire(1, rows1, sem1)

    def body(i, carry):
        j = 2 * i
        drain(j, rows0, sem0)
        fire(j + 2, rows0, sem0)
        drain(j + 1, rows1, sem1)
        fire(j + 3, rows1, sem1)
        return carry

    lax.fori_loop(0, NCHUNK // 2 - 1, body, 0)
    drain(NCHUNK - 2, rows0, sem0)
    drain(NCHUNK - 1, rows1, sem1)
    plsc.subcore_barrier()
    pltpu.sync_copy(acc_sh.at[sl], out_hbm.at[pl.ds(cid * NPAD + sid * RPS, RPS)])


def _norms_prescale(degs, x):
    def body(deg_ref, x_ref, nout_ref, nin_ref, g0_ref):
        d_out = deg_ref[0] + deg_ref[2]
        d_in = deg_ref[1] + deg_ref[3]
        n_out = lax.rsqrt(jnp.maximum(d_out, 1.0))[:N]
        n_in = lax.rsqrt(jnp.maximum(d_in, 1.0))[:N]
        nout_ref[...] = n_out
        nin_ref[...] = n_in
        g0_ref[...] = x_ref[...] * n_out

    return pl.pallas_call(
        body,
        out_shape=[
            jax.ShapeDtypeStruct((N, 1), jnp.float32),
            jax.ShapeDtypeStruct((N, 1), jnp.float32),
            jax.ShapeDtypeStruct((N, D), jnp.float32),
        ],
    )(degs, x)


def _layer_mid(p, n_in, n_out, w, b):
    def body(p_ref, nin_ref, nout_ref, w_ref, b_ref, h_ref, g_ref):
        agg = (p_ref[pl.ds(0, N), :] + p_ref[pl.ds(NPAD, N), :]) * nin_ref[...]
        t = jnp.dot(agg, w_ref[...], preferred_element_type=jnp.float32)
        h = jnp.maximum(t + b_ref[...], 0.0)
        h_ref[...] = h
        g_ref[...] = h * nout_ref[...]

    return pl.pallas_call(
        body,
        out_shape=[
            jax.ShapeDtypeStruct((N, D), jnp.float32),
            jax.ShapeDtypeStruct((N, D), jnp.float32),
        ],
    )(p, n_in, n_out, w, b)


def _layer_out(p, n_in, w, b):
    def body(p_ref, nin_ref, w_ref, b_ref, h_ref):
        agg = (p_ref[pl.ds(0, N), :] + p_ref[pl.ds(NPAD, N), :]) * nin_ref[...]
        t = jnp.dot(agg, w_ref[...], preferred_element_type=jnp.float32)
        h_ref[...] = t + b_ref[...]

    return pl.pallas_call(
        body,
        out_shape=jax.ShapeDtypeStruct((N, D), jnp.float32),
    )(p, n_in, w, b)


def kernel(x, edge_index, W1, b1, W2, b2):
    src3 = edge_index[0].reshape(NW, NCHUNK, C)
    dst3 = edge_index[1].reshape(NW, NCHUNK, C)
    pad = jnp.full((2 * 7680,), NPAD - 1, jnp.int32)
    cat = jnp.concatenate([edge_index[0], edge_index[1] + NPAD, pad])
    idx3 = cat.reshape(NW, CDCH, CDC)
    degs = _degrees(idx3).reshape(NC * 2, NPAD, 1)
    n_out, n_in, g0 = _norms_prescale(degs, x)
    b1r = b1.reshape(1, D)
    b2r = b2.reshape(1, D)
    p = _propagate(g0, src3, dst3)
    h1, g1 = _layer_mid(p, n_in, n_out, W1, b1r)
    q = _propagate(g1, src3, dst3)
    h2 = _layer_out(q, n_in, W2, b2r)
    return (h1, h2)


# speedup vs baseline: 3.0926x; 1.0096x over previous
import functools

import jax
import jax.numpy as jnp
from jax import lax
from jax.experimental import pallas as pl
from jax.experimental.pallas import tpu as pltpu
from jax.experimental.pallas import tpu_sc as plsc

N = 10000
D = 128
E = 320000
NC = 2
NS = 16
NW = NC * NS
EP = E // NW
C = 100
NCHUNK = EP // C
CDCH = 160
CDC = 128
NPAD = 10240
RPS = NPAD // NS
assert E == NW * NCHUNK * C and NCHUNK % 2 == 0
assert NW * CDCH * CDC == 2 * (E + 7680)

_mesh = plsc.VectorSubcoreMesh(
    core_axis_name="c", subcore_axis_name="s", num_cores=NC, num_subcores=NS)


@functools.partial(
    pl.kernel,
    out_type=jax.ShapeDtypeStruct((NC * 2 * NPAD,), jnp.float32),
    mesh=_mesh,
    compiler_params=pltpu.CompilerParams(use_tc_tiling_on_sc=False),
    scratch_types=[
        pltpu.VMEM((CDCH, CDC), jnp.int32),
        pltpu.VMEM((CDC,), jnp.float32),
        pltpu.VMEM((2 * RPS,), jnp.float32),
        pltpu.VMEM_SHARED((2 * NPAD,), jnp.float32),
    ],
)
def _degrees(idx_hbm, out_hbm, idx_v, ones_v, zz_v, hist_sh):
    cid = lax.axis_index("c")
    sid = lax.axis_index("s")
    wid = cid * NS + sid
    for i in range(CDC // 16):
        ones_v[pl.ds(i * 16, 16)] = jnp.ones((16,), jnp.float32)

    def zfill(i, carry):
        zz_v[pl.ds(i * 16, 16)] = jnp.zeros((16,), jnp.float32)
        return carry

    lax.fori_loop(0, 2 * RPS // 16, zfill, 0)
    pltpu.sync_copy(zz_v, hist_sh.at[pl.ds(sid * 2 * RPS, 2 * RPS)])
    pltpu.sync_copy(idx_hbm.at[wid], idx_v)
    plsc.subcore_barrier()

    def body(j, carry):
        pltpu.sync_copy(ones_v, hist_sh.at[idx_v.at[j]], add=True)
        return carry

    lax.fori_loop(0, CDCH, body, 0)
    plsc.subcore_barrier()
    sl = pl.ds(sid * 2 * RPS, 2 * RPS)
    pltpu.sync_copy(hist_sh.at[sl],
                    out_hbm.at[pl.ds(cid * 2 * NPAD + sid * 2 * RPS, 2 * RPS)])


@functools.partial(
    pl.kernel,
    out_type=jax.ShapeDtypeStruct((NC * NPAD, D), jnp.float32),
    mesh=_mesh,
    compiler_params=pltpu.CompilerParams(use_tc_tiling_on_sc=False),
    scratch_types=[
        pltpu.VMEM((NCHUNK, C), jnp.int32),
        pltpu.VMEM((NCHUNK, C), jnp.int32),
        pltpu.VMEM((C, D), jnp.float32),
        pltpu.VMEM((C, D), jnp.float32),
        pltpu.VMEM((16, D), jnp.float32),
        pltpu.VMEM_SHARED((NPAD, D), jnp.float32),
        pltpu.SemaphoreType.DMA,
        pltpu.SemaphoreType.DMA,
    ],
)
def _propagate(g_hbm, src_hbm, dst_hbm, out_hbm, src_v, dst_v, rows0, rows1,
               zz_v, acc_sh, sem0, sem1):
    cid = lax.axis_index("c")
    sid = lax.axis_index("s")
    wid = cid * NS + sid
    sl = pl.ds(sid * RPS, RPS)

    def fire(j, rows, sem):
        pltpu.make_async_copy(g_hbm.at[src_v.at[j]], rows, sem).start()

    def drain(j, rows, sem):
        pltpu.make_async_copy(g_hbm.at[src_v.at[j]], rows, sem).wait()
        pltpu.sync_copy(rows, acc_sh.at[dst_v.at[j]], add=True)

    pltpu.sync_copy(src_hbm.at[wid], src_v)
    pltpu.sync_copy(dst_hbm.at[wid], dst_v)
    fire(0, rows0, sem0)
    fire(1, rows1, sem1)
    zz_v[...] = jnp.zeros((16, D), jnp.float32)

    def zbody(r, carry):
        pltpu.sync_copy(zz_v, acc_sh.at[pl.ds(sid * RPS + r * 16, 16)])
        return carry

    lax.fori_loop(0, RPS // 16, zbody, 0)
    plsc.subcore_barrier()

    def body(i, carry):
        j = 2 * i
        drain(j, rows0, sem0)
        fire(j + 2, rows0, sem0)
        drain(j + 1, rows1, sem1)
        fire(j + 3, rows1, sem1)
        return carry

    lax.fori_loop(0, NCHUNK // 2 - 1, body, 0)
    drain(NCHUNK - 2, rows0, sem0)
    drain(NCHUNK - 1, rows1, sem1)
    plsc.subcore_barrier()
    pltpu.sync_copy(acc_sh.at[sl], out_hbm.at[pl.ds(cid * NPAD + sid * RPS, RPS)])


def _norms_prescale(degs, x):
    def body(deg_ref, x_ref, nout_ref, nin_ref, g0_ref):
        d_out = deg_ref[0] + deg_ref[2]
        d_in = deg_ref[1] + deg_ref[3]
        n_out = lax.rsqrt(jnp.maximum(d_out, 1.0))[:N]
        n_in = lax.rsqrt(jnp.maximum(d_in, 1.0))[:N]
        nout_ref[...] = n_out
        nin_ref[...] = n_in
        g0_ref[...] = x_ref[...] * n_out

    return pl.pallas_call(
        body,
        out_shape=[
            jax.ShapeDtypeStruct((N, 1), jnp.float32),
            jax.ShapeDtypeStruct((N, 1), jnp.float32),
            jax.ShapeDtypeStruct((N, D), jnp.float32),
        ],
    )(degs, x)


def _layer_mid(p, n_in, n_out, w, b):
    def body(p_ref, nin_ref, nout_ref, w_ref, b_ref, h_ref, g_ref):
        agg = (p_ref[pl.ds(0, N), :] + p_ref[pl.ds(NPAD, N), :]) * nin_ref[...]
        t = jnp.dot(agg, w_ref[...], preferred_element_type=jnp.float32)
        h = jnp.maximum(t + b_ref[...], 0.0)
        h_ref[...] = h
        g_ref[...] = h * nout_ref[...]

    return pl.pallas_call(
        body,
        out_shape=[
            jax.ShapeDtypeStruct((N, D), jnp.float32),
            jax.ShapeDtypeStruct((N, D), jnp.float32),
        ],
    )(p, n_in, n_out, w, b)


def _layer_out(p, n_in, w, b):
    def body(p_ref, nin_ref, w_ref, b_ref, h_ref):
        agg = (p_ref[pl.ds(0, N), :] + p_ref[pl.ds(NPAD, N), :]) * nin_ref[...]
        t = jnp.dot(agg, w_ref[...], preferred_element_type=jnp.float32)
        h_ref[...] = t + b_ref[...]

    return pl.pallas_call(
        body,
        out_shape=jax.ShapeDtypeStruct((N, D), jnp.float32),
    )(p, n_in, w, b)


def kernel(x, edge_index, W1, b1, W2, b2):
    src3 = edge_index[0].reshape(NW, NCHUNK, C)
    dst3 = edge_index[1].reshape(NW, NCHUNK, C)
    pad = jnp.full((2 * 7680,), NPAD - 1, jnp.int32)
    cat = jnp.concatenate([edge_index[0], edge_index[1] + NPAD, pad])
    idx3 = cat.reshape(NW, CDCH, CDC)
    degs = _degrees(idx3).reshape(NC * 2, NPAD, 1)
    n_out, n_in, g0 = _norms_prescale(degs, x)
    b1r = b1.reshape(1, D)
    b2r = b2.reshape(1, D)
    p = _propagate(g0, src3, dst3)
    h1, g1 = _layer_mid(p, n_in, n_out, W1, b1r)
    q = _propagate(g1, src3, dst3)
    h2 = _layer_out(q, n_in, W2, b2r)
    return (h1, h2)
